# baseline (speedup 1.0000x reference)
import jax
import jax.numpy as jnp
from jax import lax
from jax.experimental import pallas as pl
from jax.experimental.pallas import tpu as pltpu
from jax.experimental.pallas import tpu_sc as plsc

D_MODEL = 256
TABLE_ROWS = 129
SEQ_LEN = 512

T_LEN = 1016
N_COPIES = 8
FULL_LEN = N_COPIES * T_LEN
HALF_A = 504
ROWS_PER_WORKER = SEQ_LEN // 32


def _build_full(tbl_ref, full_ref):
    t = tbl_ref[...]
    t0 = t[0:1]
    t128 = t[TABLE_ROWS - 1:TABLE_ROWS]
    pieces = []
    for r in range(1, N_COPIES + 1):
        pieces.append(jnp.broadcast_to(t0, (448 - r, D_MODEL)))
        pieces.append(t)
        pieces.append(jnp.broadcast_to(t128, (T_LEN - 577 + r, D_MODEL)))
    full_ref[...] = jnp.concatenate(pieces, axis=0)


def _sc_body(tfull_hbm, out_hbm, t_all, sem):
    c = lax.axis_index("c")
    s = lax.axis_index("s")

    r2 = s // 2

    @pl.when(s % 2 == 0)
    def _():
        off = pl.multiple_of(r2 * T_LEN, 8)
        pltpu.sync_copy(tfull_hbm.at[pl.ds(off, HALF_A)],
                        t_all.at[pl.ds(off, HALF_A)])

    @pl.when(s % 2 == 1)
    def _():
        off = pl.multiple_of(r2 * T_LEN + HALF_A, 8)
        pltpu.sync_copy(tfull_hbm.at[pl.ds(off, T_LEN - HALF_A)],
                        t_all.at[pl.ds(off, T_LEN - HALF_A)])

    plsc.subcore_barrier()

    w = c * 16 + s
    copies = []
    for k in range(ROWS_PER_WORKER):
        i = 16 * w + k
        r = N_COPIES - (k % 8)
        a0 = 504 - 16 * w - 8 * (k // 8)
        off = pl.multiple_of((r - 1) * T_LEN + a0, 8)
        src_ref = tfull_hbm if k % 4 == 1 else t_all
        copies.append(
            pltpu.async_copy(src_ref.at[pl.ds(off, SEQ_LEN)],
                             out_hbm.at[i], sem))
    for cp in copies:
        cp.wait()


def kernel(x, embeddings_table):
    table = embeddings_table.astype(jnp.float32)
    tfull = pl.pallas_call(
        _build_full,
        out_shape=jax.ShapeDtypeStruct((FULL_LEN, D_MODEL), jnp.float32),
    )(table)

    mesh = plsc.VectorSubcoreMesh(core_axis_name="c", subcore_axis_name="s")
    run = pl.kernel(
        _sc_body,
        out_type=jax.ShapeDtypeStruct((SEQ_LEN, SEQ_LEN, D_MODEL), jnp.float32),
        mesh=mesh,
        scratch_types=[
            pltpu.VMEM_SHARED((FULL_LEN, D_MODEL), jnp.float32),
            pltpu.SemaphoreType.DMA,
        ],
    )
    out = run(tfull)
    return out.astype(x.dtype)


# speedup vs baseline: 12.5358x; 12.5358x over previous
import jax
import jax.numpy as jnp
from jax import lax
from jax.experimental import pallas as pl
from jax.experimental.pallas import tpu as pltpu
from jax.experimental.pallas import tpu_sc as plsc

D_MODEL = 256
TABLE_ROWS = 129
SEQ_LEN = 512

T_LEN = 1016
N_COPIES = 8
REP_LEN = 440
MID_LO, MID_HI = 440, 576
MID_LEN = MID_HI - MID_LO
MID_SPLIT = 64
PARTS_MIDS = 2 * REP_LEN
PARTS_LEN = 2048
ROWS_PER_WORKER = SEQ_LEN // 32


def _build_parts(tbl_ref, parts_ref):
    t = tbl_ref[...]
    t0 = t[0:1]
    t128 = t[TABLE_ROWS - 1:TABLE_ROWS]
    pieces = [
        jnp.broadcast_to(t0, (REP_LEN, D_MODEL)),
        jnp.broadcast_to(t128, (REP_LEN, D_MODEL)),
    ]
    for r in range(1, N_COPIES + 1):
        if 8 - r:
            pieces.append(jnp.broadcast_to(t0, (8 - r, D_MODEL)))
        pieces.append(t)
        if r - 1:
            pieces.append(jnp.broadcast_to(t128, (r - 1, D_MODEL)))
    used = PARTS_MIDS + N_COPIES * MID_LEN
    pieces.append(jnp.broadcast_to(t128, (PARTS_LEN - used, D_MODEL)))
    parts_ref[...] = jnp.concatenate(pieces, axis=0)


def _sc_body(parts_hbm, out_hbm, t_all, sem):
    c = lax.axis_index("c")
    s = lax.axis_index("s")

    @pl.when(s < N_COPIES)
    def _():
        base = pl.multiple_of(s * T_LEN, 8)
        cp0 = pltpu.async_copy(parts_hbm.at[pl.ds(0, REP_LEN)],
                               t_all.at[pl.ds(base, REP_LEN)], sem)
        src_mid = pl.multiple_of(PARTS_MIDS + s * MID_LEN, 8)
        dst_mid = pl.multiple_of(s * T_LEN + MID_LO, 8)
        cp1 = pltpu.async_copy(parts_hbm.at[pl.ds(src_mid, MID_SPLIT)],
                               t_all.at[pl.ds(dst_mid, MID_SPLIT)], sem)
        cp0.wait()
        cp1.wait()

    @pl.when(s >= N_COPIES)
    def _():
        s8 = s - N_COPIES
        src_mid = pl.multiple_of(PARTS_MIDS + s8 * MID_LEN + MID_SPLIT, 8)
        dst_mid = pl.multiple_of(s8 * T_LEN + MID_LO + MID_SPLIT, 8)
        cp0 = pltpu.async_copy(
            parts_hbm.at[pl.ds(src_mid, MID_LEN - MID_SPLIT)],
            t_all.at[pl.ds(dst_mid, MID_LEN - MID_SPLIT)], sem)
        dst = pl.multiple_of(s8 * T_LEN + MID_HI, 8)
        cp1 = pltpu.async_copy(parts_hbm.at[pl.ds(REP_LEN, REP_LEN)],
                               t_all.at[pl.ds(dst, REP_LEN)], sem)
        cp0.wait()
        cp1.wait()

    plsc.subcore_barrier()

    w = c * 16 + s
    copies = []
    for k in range(ROWS_PER_WORKER):
        i = 16 * w + k
        r = N_COPIES - (k % 8)
        a0 = 504 - 16 * w - 8 * (k // 8)
        off = pl.multiple_of((r - 1) * T_LEN + a0, 8)
        copies.append(
            pltpu.async_copy(t_all.at[pl.ds(off, SEQ_LEN)],
                             out_hbm.at[i], sem))
    for cp in copies:
        cp.wait()


def kernel(x, embeddings_table):
    table = embeddings_table.astype(jnp.float32)
    parts = pl.pallas_call(
        _build_parts,
        out_shape=jax.ShapeDtypeStruct((PARTS_LEN, D_MODEL), jnp.float32),
    )(table)

    mesh = plsc.VectorSubcoreMesh(core_axis_name="c", subcore_axis_name="s")
    run = pl.kernel(
        _sc_body,
        out_type=jax.ShapeDtypeStruct((SEQ_LEN, SEQ_LEN, D_MODEL), jnp.float32),
        mesh=mesh,
        scratch_types=[
            pltpu.VMEM_SHARED((N_COPIES * T_LEN, D_MODEL), jnp.float32),
            pltpu.SemaphoreType.DMA,
        ],
    )
    out = run(parts)
    return out.astype(x.dtype)
